# baseline (speedup 1.0000x reference)
import functools

import jax
import jax.numpy as jnp
from jax.experimental import pallas as pl

_VOCABS = (6, 4, 4, 4, 6, 4, 4)
_OFF = (0, 6, 10, 14, 18, 24, 28)
_B = 16384
_BS = 2048


def _body(uc_ref, un_ref, t0, t1, t2, t3, t4, t5, t6, w1_ref, b1_ref, w2_ref,
          b2_ref, w3_ref, b3_ref, out_ref):
    f32 = jnp.float32
    l8 = jax.lax.broadcasted_iota(jnp.int32, (8, 32), 1)
    s8 = jax.lax.broadcasted_iota(jnp.int32, (8, 32), 0)
    fv = jnp.zeros((8, 32), jnp.int32)
    offv = jnp.zeros((8, 32), jnp.int32)
    for bnd, jump in zip(_OFF[1:], (6, 4, 4, 4, 6, 4)):
        step = (l8 >= bnd).astype(jnp.int32)
        fv = fv + step
        offv = offv + jump * step
    rm = (fv == s8).astype(f32)
    cmpv = (l8 - offv).astype(f32)[0:1]

    ucf = uc_ref[...].astype(f32)
    ucx = jnp.dot(ucf, rm[:7, :], preferred_element_type=f32)
    m = (ucx == cmpv).astype(f32)

    tabs = (t0, t1, t2, t3, t4, t5, t6)
    g = jnp.concatenate(
        [jnp.dot(t[...], w1_ref[8 * i:8 * i + 8, :],
                 preferred_element_type=f32) for i, t in enumerate(tabs)],
        axis=0)
    h = (jnp.dot(m, g, preferred_element_type=f32)
         + jnp.dot(un_ref[...], w1_ref[56:58, :], preferred_element_type=f32)
         + b1_ref[...])
    h = jnp.maximum(h, 0.0)
    h = jnp.dot(h, w2_ref[...], preferred_element_type=f32) + b2_ref[...]
    h = jnp.maximum(h, 0.0)
    out_ref[...] = (jnp.dot(h, w3_ref[...], preferred_element_type=f32)
                    + b3_ref[...])


@functools.partial(jax.jit, static_argnames=("interpret",))
def kernel(u_cat, u_num, T_light, T_hum, T_care, T_size, T_climate, T_water,
           T_care_freq, W1, b1, W2, b2, W3, b3, interpret=False):
    tables = [T_light, T_hum, T_care, T_size, T_climate, T_water, T_care_freq]
    const = lambda s: pl.BlockSpec(s, lambda i: (0,) * len(s))
    grid = (_B // _BS,)
    out = pl.pallas_call(
        _body,
        grid=grid,
        in_specs=[
            pl.BlockSpec((_BS, 7), lambda i: (i, 0)),
            pl.BlockSpec((_BS, 2), lambda i: (i, 0)),
            *[const((v, 8)) for v in _VOCABS],
            const((58, 128)),
            const((1, 128)),
            const((128, 128)),
            const((1, 128)),
            const((128, 64)),
            const((1, 64)),
        ],
        out_specs=pl.BlockSpec((_BS, 64), lambda i: (i, 0)),
        out_shape=jax.ShapeDtypeStruct((_B, 64), jnp.float32),
        interpret=interpret,
    )(u_cat.astype(jnp.int32), u_num, *tables, W1, b1.reshape(1, 128), W2,
      b2.reshape(1, 128), W3, b3.reshape(1, 64))
    return out


# speedup vs baseline: 1.0996x; 1.0996x over previous
import functools

import jax
import jax.numpy as jnp
from jax.experimental import pallas as pl

_VOCABS = (6, 4, 4, 4, 6, 4, 4)
_OFF = (0, 6, 10, 14, 18, 24, 28)
_B = 16384
_BS = 4096


def _body(uc_ref, un_ref, t0, t1, t2, t3, t4, t5, t6, w1_ref, b1_ref, w2_ref,
          b2_ref, w3_ref, b3_ref, out_ref):
    f32 = jnp.float32
    l8 = jax.lax.broadcasted_iota(jnp.int32, (8, 32), 1)
    s8 = jax.lax.broadcasted_iota(jnp.int32, (8, 32), 0)
    fv = jnp.zeros((8, 32), jnp.int32)
    offv = jnp.zeros((8, 32), jnp.int32)
    for bnd, jump in zip(_OFF[1:], (6, 4, 4, 4, 6, 4)):
        step = (l8 >= bnd).astype(jnp.int32)
        fv = fv + step
        offv = offv + jump * step
    rm = (fv == s8).astype(f32)
    cmpv = (l8 - offv).astype(f32)[0:1]

    ucf = uc_ref[...].astype(f32)
    ucx = jnp.dot(ucf, rm[:7, :], preferred_element_type=f32)
    m = (ucx == cmpv).astype(f32)

    tabs = (t0, t1, t2, t3, t4, t5, t6)
    g = jnp.concatenate(
        [jnp.dot(t[...], w1_ref[8 * i:8 * i + 8, :],
                 preferred_element_type=f32) for i, t in enumerate(tabs)],
        axis=0)
    bf16 = jnp.bfloat16
    h = (jnp.dot(m.astype(bf16), g.astype(bf16), preferred_element_type=f32)
         + jnp.dot(un_ref[...], w1_ref[56:58, :], preferred_element_type=f32)
         + b1_ref[...])
    h = jnp.maximum(h, 0.0)
    h = (jnp.dot(h.astype(bf16), w2_ref[...].astype(bf16),
                 preferred_element_type=f32) + b2_ref[...])
    h = jnp.maximum(h, 0.0)
    out_ref[...] = (jnp.dot(h.astype(bf16), w3_ref[...].astype(bf16),
                            preferred_element_type=f32) + b3_ref[...])


@functools.partial(jax.jit, static_argnames=("interpret",))
def kernel(u_cat, u_num, T_light, T_hum, T_care, T_size, T_climate, T_water,
           T_care_freq, W1, b1, W2, b2, W3, b3, interpret=False):
    tables = [T_light, T_hum, T_care, T_size, T_climate, T_water, T_care_freq]
    const = lambda s: pl.BlockSpec(s, lambda i: (0,) * len(s))
    grid = (_B // _BS,)
    out = pl.pallas_call(
        _body,
        grid=grid,
        in_specs=[
            pl.BlockSpec((_BS, 7), lambda i: (i, 0)),
            pl.BlockSpec((_BS, 2), lambda i: (i, 0)),
            *[const((v, 8)) for v in _VOCABS],
            const((58, 128)),
            const((1, 128)),
            const((128, 128)),
            const((1, 128)),
            const((128, 64)),
            const((1, 64)),
        ],
        out_specs=pl.BlockSpec((_BS, 64), lambda i: (i, 0)),
        out_shape=jax.ShapeDtypeStruct((_B, 64), jnp.float32),
        interpret=interpret,
    )(u_cat.astype(jnp.int32), u_num, *tables, W1, b1.reshape(1, 128), W2,
      b2.reshape(1, 128), W3, b3.reshape(1, 64))
    return out


# speedup vs baseline: 1.1023x; 1.0025x over previous
import functools

import jax
import jax.numpy as jnp
from jax.experimental import pallas as pl

_VOCABS = (6, 4, 4, 4, 6, 4, 4)
_OFF = (0, 6, 10, 14, 18, 24, 28)
_B = 16384
_BS = 4096


def _body(uc_ref, un_ref, t0, t1, t2, t3, t4, t5, t6, w1_ref, b1_ref, w2_ref,
          b2_ref, w3_ref, b3_ref, out_ref):
    f32 = jnp.float32
    l8 = jax.lax.broadcasted_iota(jnp.int32, (8, 32), 1)
    s8 = jax.lax.broadcasted_iota(jnp.int32, (8, 32), 0)
    fv = jnp.zeros((8, 32), jnp.int32)
    offv = jnp.zeros((8, 32), jnp.int32)
    for bnd, jump in zip(_OFF[1:], (6, 4, 4, 4, 6, 4)):
        step = (l8 >= bnd).astype(jnp.int32)
        fv = fv + step
        offv = offv + jump * step
    rm = (fv == s8).astype(f32)
    cmpv = (l8 - offv).astype(f32)[0:1]

    ucf = uc_ref[...].astype(f32)
    ucx = jnp.dot(ucf, rm[:7, :], preferred_element_type=f32)
    m = (ucx == cmpv).astype(f32)

    tabs = (t0, t1, t2, t3, t4, t5, t6)
    g = jnp.concatenate(
        [jnp.dot(t[...], w1_ref[8 * i:8 * i + 8, :],
                 preferred_element_type=f32) for i, t in enumerate(tabs)],
        axis=0)
    bf16 = jnp.bfloat16
    b1 = b1_ref[...].reshape(1, 128)
    b2 = b2_ref[...].reshape(1, 128)
    b3 = b3_ref[...].reshape(1, 64)
    h = (jnp.dot(m.astype(bf16), g.astype(bf16), preferred_element_type=f32)
         + jnp.dot(un_ref[...], w1_ref[56:58, :], preferred_element_type=f32)
         + b1)
    h = jnp.maximum(h, 0.0)
    h = (jnp.dot(h.astype(bf16), w2_ref[...].astype(bf16),
                 preferred_element_type=f32) + b2)
    h = jnp.maximum(h, 0.0)
    out_ref[...] = (jnp.dot(h.astype(bf16), w3_ref[...].astype(bf16),
                            preferred_element_type=f32) + b3)


@functools.partial(jax.jit, static_argnames=("interpret",))
def kernel(u_cat, u_num, T_light, T_hum, T_care, T_size, T_climate, T_water,
           T_care_freq, W1, b1, W2, b2, W3, b3, interpret=False):
    tables = [T_light, T_hum, T_care, T_size, T_climate, T_water, T_care_freq]
    const = lambda s: pl.BlockSpec(s, lambda i: (0,) * len(s))
    grid = (_B // _BS,)
    out = pl.pallas_call(
        _body,
        grid=grid,
        in_specs=[
            pl.BlockSpec((_BS, 7), lambda i: (i, 0)),
            pl.BlockSpec((_BS, 2), lambda i: (i, 0)),
            *[const((v, 8)) for v in _VOCABS],
            const((58, 128)),
            const((128,)),
            const((128, 128)),
            const((128,)),
            const((128, 64)),
            const((64,)),
        ],
        out_specs=pl.BlockSpec((_BS, 64), lambda i: (i, 0)),
        out_shape=jax.ShapeDtypeStruct((_B, 64), jnp.float32),
        interpret=interpret,
    )(u_cat.astype(jnp.int32), u_num, *tables, W1, b1, W2, b2, W3, b3)
    return out


# speedup vs baseline: 4.6215x; 4.1926x over previous
import functools

import jax
import jax.numpy as jnp
from jax.experimental import pallas as pl

_VOCABS = (6, 4, 4, 4, 6, 4, 4)
_OFF = (0, 6, 10, 14, 18, 24, 28)
_B = 16384
_BS = 4096

_DT0 = (((0,), (0,)), ((), ()))


def _body(uct_ref, unt_ref, t0, t1, t2, t3, t4, t5, t6, w1_ref, b1_ref,
          w2_ref, b2_ref, w3t_ref, b3_ref, out_ref):
    f32 = jnp.float32
    bf16 = jnp.bfloat16
    bs = out_ref.shape[1]

    s32 = jax.lax.broadcasted_iota(jnp.int32, (32, 8), 0)
    l32 = jax.lax.broadcasted_iota(jnp.int32, (32, 8), 1)
    fv = jnp.zeros((32, 8), jnp.int32)
    offv = jnp.zeros((32, 8), jnp.int32)
    for bnd, jump in zip(_OFF[1:], (6, 4, 4, 4, 6, 4)):
        step = (s32 >= bnd).astype(jnp.int32)
        fv = fv + step
        offv = offv + jump * step
    rt = (fv == l32).astype(f32)
    cmp_col = (s32 - offv).astype(f32)[:, 0:1]

    uctf = uct_ref[...].astype(f32)
    uct8 = jnp.concatenate([uctf, jnp.zeros((1, bs), f32)], axis=0)
    ucx_t = jnp.dot(rt, uct8, preferred_element_type=f32)
    mt = (ucx_t == cmp_col).astype(bf16)

    tabs = (t0, t1, t2, t3, t4, t5, t6)
    g = jnp.concatenate(
        [jnp.dot(t[...], w1_ref[8 * i:8 * i + 8, :],
                 preferred_element_type=f32) for i, t in enumerate(tabs)],
        axis=0)

    ones_row = jnp.ones((1, bs), f32)
    b1c = jax.lax.dot_general(b1_ref[...].reshape(1, 128), ones_row, _DT0,
                              preferred_element_type=f32)
    b2c = jax.lax.dot_general(b2_ref[...].reshape(1, 128), ones_row, _DT0,
                              preferred_element_type=f32)
    b3c = jax.lax.dot_general(b3_ref[...].reshape(1, 64), ones_row, _DT0,
                              preferred_element_type=f32)

    h = (jax.lax.dot_general(g.astype(bf16), mt, _DT0,
                             preferred_element_type=f32)
         + jax.lax.dot_general(w1_ref[56:58, :], unt_ref[...], _DT0,
                               preferred_element_type=f32)
         + b1c)
    h = jnp.maximum(h, 0.0)
    h = jax.lax.dot_general(w2_ref[...].astype(bf16), h.astype(bf16), _DT0,
                            preferred_element_type=f32) + b2c
    h = jnp.maximum(h, 0.0)
    out_ref[...] = (jnp.dot(w3t_ref[...].astype(bf16), h.astype(bf16),
                            preferred_element_type=f32) + b3c)


@functools.partial(jax.jit, static_argnames=("interpret",))
def kernel(u_cat, u_num, T_light, T_hum, T_care, T_size, T_climate, T_water,
           T_care_freq, W1, b1, W2, b2, W3, b3, interpret=False):
    tables = [T_light, T_hum, T_care, T_size, T_climate, T_water, T_care_freq]
    const = lambda s: pl.BlockSpec(s, lambda i: (0,) * len(s))
    grid = (_B // _BS,)
    out_t = pl.pallas_call(
        _body,
        grid=grid,
        in_specs=[
            pl.BlockSpec((7, _BS), lambda i: (0, i)),
            pl.BlockSpec((2, _BS), lambda i: (0, i)),
            *[const((v, 8)) for v in _VOCABS],
            const((58, 128)),
            const((128,)),
            const((128, 128)),
            const((128,)),
            const((64, 128)),
            const((64,)),
        ],
        out_specs=pl.BlockSpec((64, _BS), lambda i: (0, i)),
        out_shape=jax.ShapeDtypeStruct((64, _B), jnp.float32),
        interpret=interpret,
    )(u_cat.astype(jnp.int32).T, u_num.T, *tables, W1, b1, W2, b2, W3.T, b3)
    return out_t.T


# speedup vs baseline: 4.7099x; 1.0191x over previous
import functools

import jax
import jax.numpy as jnp
from jax.experimental import pallas as pl

_VOCABS = (6, 4, 4, 4, 6, 4, 4)
_OFF = (0, 6, 10, 14, 18, 24, 28)
_B = 16384
_BS = 8192

_DT0 = (((0,), (0,)), ((), ()))


def _body(uct_ref, unt_ref, t0, t1, t2, t3, t4, t5, t6, w1_ref, b1_ref,
          w2_ref, b2_ref, w3t_ref, b3_ref, out_ref):
    f32 = jnp.float32
    bf16 = jnp.bfloat16
    bs = out_ref.shape[1]

    s32 = jax.lax.broadcasted_iota(jnp.int32, (32, 8), 0)
    l32 = jax.lax.broadcasted_iota(jnp.int32, (32, 8), 1)
    fv = jnp.zeros((32, 8), jnp.int32)
    offv = jnp.zeros((32, 8), jnp.int32)
    for bnd, jump in zip(_OFF[1:], (6, 4, 4, 4, 6, 4)):
        step = (s32 >= bnd).astype(jnp.int32)
        fv = fv + step
        offv = offv + jump * step
    rt = (fv == l32).astype(f32)
    cmp_col = (s32 - offv).astype(f32)[:, 0:1]

    uctf = uct_ref[...].astype(f32)
    uct8 = jnp.concatenate([uctf, jnp.zeros((1, bs), f32)], axis=0)
    ucx_t = jnp.dot(rt, uct8, preferred_element_type=f32)
    mt = (ucx_t == cmp_col).astype(bf16)

    tabs = (t0, t1, t2, t3, t4, t5, t6)
    g = jnp.concatenate(
        [jnp.dot(t[...], w1_ref[8 * i:8 * i + 8, :],
                 preferred_element_type=f32) for i, t in enumerate(tabs)],
        axis=0)

    ones_row = jnp.ones((1, bs), f32)
    b1c = jax.lax.dot_general(b1_ref[...].reshape(1, 128), ones_row, _DT0,
                              preferred_element_type=f32)
    b2c = jax.lax.dot_general(b2_ref[...].reshape(1, 128), ones_row, _DT0,
                              preferred_element_type=f32)
    b3c = jax.lax.dot_general(b3_ref[...].reshape(1, 64), ones_row, _DT0,
                              preferred_element_type=f32)

    h = (jax.lax.dot_general(g.astype(bf16), mt, _DT0,
                             preferred_element_type=f32)
         + jax.lax.dot_general(w1_ref[56:58, :], unt_ref[...], _DT0,
                               preferred_element_type=f32)
         + b1c)
    h = jnp.maximum(h, 0.0)
    h = jax.lax.dot_general(w2_ref[...].astype(bf16), h.astype(bf16), _DT0,
                            preferred_element_type=f32) + b2c
    h = jnp.maximum(h, 0.0)
    out_ref[...] = (jnp.dot(w3t_ref[...].astype(bf16), h.astype(bf16),
                            preferred_element_type=f32) + b3c)


@functools.partial(jax.jit, static_argnames=("interpret",))
def kernel(u_cat, u_num, T_light, T_hum, T_care, T_size, T_climate, T_water,
           T_care_freq, W1, b1, W2, b2, W3, b3, interpret=False):
    tables = [T_light, T_hum, T_care, T_size, T_climate, T_water, T_care_freq]
    const = lambda s: pl.BlockSpec(s, lambda i: (0,) * len(s))
    grid = (_B // _BS,)
    out_t = pl.pallas_call(
        _body,
        grid=grid,
        in_specs=[
            pl.BlockSpec((7, _BS), lambda i: (0, i)),
            pl.BlockSpec((2, _BS), lambda i: (0, i)),
            *[const((v, 8)) for v in _VOCABS],
            const((58, 128)),
            const((128,)),
            const((128, 128)),
            const((128,)),
            const((64, 128)),
            const((64,)),
        ],
        out_specs=pl.BlockSpec((64, _BS), lambda i: (0, i)),
        out_shape=jax.ShapeDtypeStruct((64, _B), jnp.float32),
        interpret=interpret,
    )(u_cat.astype(jnp.int32).T, u_num.T, *tables, W1, b1, W2, b2, W3.T, b3)
    return out_t.T
